# baseline (speedup 1.0000x reference)
import numpy as np
import jax
import jax.numpy as jnp
from jax.experimental import pallas as pl
from jax.experimental.pallas import tpu as pltpu

_B = 128
_N = 100000
_CHUNK = 12800
_GRID = (_N + _CHUNK - 1) // _CHUNK

_TINY = np.float32(np.finfo(np.float32).tiny)
_NEG_INF = np.float32(-np.inf)


def _threefry2x32(x0, x1):
    ks0 = np.uint32(0)
    ks1 = np.uint32(42)
    ks2 = np.uint32(ks0 ^ ks1 ^ np.uint32(0x1BD11BDA))

    def rotl(x, r):
        return (x << np.uint32(r)) | (x >> np.uint32(32 - r))

    x0 = x0 + ks0
    x1 = x1 + ks1
    rots = ((13, 15, 26, 6), (17, 29, 16, 24))
    inject = ((ks1, ks2, 1), (ks2, ks0, 2), (ks0, ks1, 3),
              (ks1, ks2, 4), (ks2, ks0, 5))
    for grp in range(5):
        for r in rots[grp % 2]:
            x0 = x0 + x1
            x1 = rotl(x1, r) ^ x0
        a, b, c = inject[grp]
        x0 = x0 + a
        x1 = x1 + b + np.uint32(c)
    return x0, x1


def _sample_kernel(logits_ref, out_ref, best_val, best_idx):
    j = pl.program_id(0)

    col = jax.lax.broadcasted_iota(jnp.int32, (_B, _CHUNK), 1) + j * _CHUNK
    row = jax.lax.broadcasted_iota(jnp.int32, (_B, _CHUNK), 0)
    flat = (row * _N + col).astype(jnp.uint32)

    y0, y1 = _threefry2x32(jnp.zeros_like(flat), flat)
    bits = y0 ^ y1

    fbits = (bits >> np.uint32(9)) | np.uint32(0x3F800000)
    floats = jax.lax.bitcast_convert_type(fbits, jnp.float32) - np.float32(1.0)
    u = jnp.maximum(_TINY, floats + _TINY)
    g = -jnp.log(-jnp.log(u))

    val = jnp.where(col < _N, logits_ref[...] + g, _NEG_INF)

    m = jnp.max(val, axis=1, keepdims=True)
    cand = jnp.where(val == m, col, np.int32(2**31 - 1))
    idx = jnp.min(cand, axis=1, keepdims=True)

    @pl.when(j == 0)
    def _init():
        best_val[...] = m
        best_idx[...] = idx

    @pl.when(j > 0)
    def _update():
        better = m > best_val[...]
        best_val[...] = jnp.where(better, m, best_val[...])
        best_idx[...] = jnp.where(better, idx, best_idx[...])

    @pl.when(j == _GRID - 1)
    def _finish():
        out_ref[...] = best_idx[...]


def kernel(logits):
    out = pl.pallas_call(
        _sample_kernel,
        grid=(_GRID,),
        in_specs=[pl.BlockSpec((_B, _CHUNK), lambda j: (0, j))],
        out_specs=pl.BlockSpec((_B, 1), lambda j: (0, 0)),
        out_shape=jax.ShapeDtypeStruct((_B, 1), jnp.int32),
        scratch_shapes=[
            pltpu.VMEM((_B, 1), jnp.float32),
            pltpu.VMEM((_B, 1), jnp.int32),
        ],
        compiler_params=pltpu.CompilerParams(
            dimension_semantics=("arbitrary",),
        ),
    )(logits)
    return out.reshape(_B)


# speedup vs baseline: 1.0258x; 1.0258x over previous
import numpy as np
import jax
import jax.numpy as jnp
from jax.experimental import pallas as pl
from jax.experimental.pallas import tpu as pltpu

_B = 128
_N = 100000
_CHUNK = 12800
_GRID = (_N + _CHUNK - 1) // _CHUNK

_TINY = np.float32(np.finfo(np.float32).tiny)

_KS1 = np.uint32(42)
_KS2 = np.uint32(42 ^ 0x1BD11BDA)


def _threefry_bits(x1):
    def rotl(x, r):
        return (x << np.uint32(r)) | (x >> np.uint32(32 - r))

    x0 = x1
    x1 = rotl(x1, 13) ^ x0
    for r in (15, 26, 6):
        x0 = x0 + x1
        x1 = rotl(x1, r) ^ x0
    x0 = x0 + _KS1
    x1 = x1 + np.uint32(_KS2 + np.uint32(1))

    for r in (17, 29, 16, 24):
        x0 = x0 + x1
        x1 = rotl(x1, r) ^ x0
    x0 = x0 + _KS2
    x1 = x1 + np.uint32(2)

    for r in (13, 15, 26, 6):
        x0 = x0 + x1
        x1 = rotl(x1, r) ^ x0
    x1 = x1 + np.uint32(_KS1 + np.uint32(3))

    for r in (17, 29, 16, 24):
        x0 = x0 + x1
        x1 = rotl(x1, r) ^ x0
    x0 = x0 + _KS1
    x1 = x1 + np.uint32(_KS2 + np.uint32(4))

    for r in (13, 15, 26, 6):
        x0 = x0 + x1
        x1 = rotl(x1, r) ^ x0
    x0 = x0 + _KS2
    x1 = x1 + np.uint32(5)

    return x0 ^ x1


def _sample_kernel(logits_ref, out_ref, best_val, best_idx):
    j = pl.program_id(0)

    row_base = (jax.lax.broadcasted_iota(jnp.int32, (_B, 1), 0) * _N
                + (j * _CHUNK + 42))
    col_local = jax.lax.broadcasted_iota(jnp.int32, (_B, _CHUNK), 1)
    x1 = (row_base + col_local).astype(jnp.uint32)

    bits = _threefry_bits(x1)

    fbits = (bits >> np.uint32(9)) | np.uint32(0x3F800000)
    floats = jax.lax.bitcast_convert_type(fbits, jnp.float32) - np.float32(1.0)
    u = jnp.maximum(_TINY, floats + _TINY)
    g = -jnp.log(-jnp.log(u))

    col = col_local + j * _CHUNK
    val = jnp.where(col < _N, logits_ref[...] + g, np.float32(-np.inf))

    m = jnp.max(val, axis=1, keepdims=True)
    cand = jnp.where(val == m, col_local, np.int32(2**31 - 1))
    idx = jnp.min(cand, axis=1, keepdims=True) + j * _CHUNK

    @pl.when(j == 0)
    def _init():
        best_val[...] = m
        best_idx[...] = idx

    @pl.when(j > 0)
    def _update():
        better = m > best_val[...]
        best_val[...] = jnp.where(better, m, best_val[...])
        best_idx[...] = jnp.where(better, idx, best_idx[...])

    @pl.when(j == _GRID - 1)
    def _finish():
        out_ref[...] = best_idx[...]


def kernel(logits):
    out = pl.pallas_call(
        _sample_kernel,
        grid=(_GRID,),
        in_specs=[pl.BlockSpec((_B, _CHUNK), lambda j: (0, j))],
        out_specs=pl.BlockSpec((_B, 1), lambda j: (0, 0)),
        out_shape=jax.ShapeDtypeStruct((_B, 1), jnp.int32),
        scratch_shapes=[
            pltpu.VMEM((_B, 1), jnp.float32),
            pltpu.VMEM((_B, 1), jnp.int32),
        ],
        compiler_params=pltpu.CompilerParams(
            dimension_semantics=("arbitrary",),
        ),
    )(logits)
    return out.reshape(_B)


# speedup vs baseline: 1.7160x; 1.6729x over previous
import numpy as np
import jax
import jax.numpy as jnp
from jax.experimental import pallas as pl
from jax.experimental.pallas import tpu as pltpu

_B = 128
_N = 100000
_CHUNK = 12800
_GRID = (_N + _CHUNK - 1) // _CHUNK
_SUB = 8
_TILE = 1280
_NSTRIP = _B // _SUB
_NTILE = _CHUNK // _TILE

_TINY = np.float32(np.finfo(np.float32).tiny)
_NEG_INF = np.float32(-np.inf)

_KS1 = np.uint32(42)
_KS2 = np.uint32(42 ^ 0x1BD11BDA)


def _threefry_bits(x1):
    def rotl(x, r):
        return (x << np.uint32(r)) | (x >> np.uint32(32 - r))

    x0 = x1
    x1 = rotl(x1, 13) ^ x0
    for r in (15, 26, 6):
        x0 = x0 + x1
        x1 = rotl(x1, r) ^ x0
    x0 = x0 + _KS1
    x1 = x1 + np.uint32(_KS2 + np.uint32(1))

    for r in (17, 29, 16, 24):
        x0 = x0 + x1
        x1 = rotl(x1, r) ^ x0
    x0 = x0 + _KS2
    x1 = x1 + np.uint32(2)

    for r in (13, 15, 26, 6):
        x0 = x0 + x1
        x1 = rotl(x1, r) ^ x0
    x1 = x1 + np.uint32(_KS1 + np.uint32(3))

    for r in (17, 29, 16, 24):
        x0 = x0 + x1
        x1 = rotl(x1, r) ^ x0
    x0 = x0 + _KS1
    x1 = x1 + np.uint32(_KS2 + np.uint32(4))

    for r in (13, 15, 26, 6):
        x0 = x0 + x1
        x1 = rotl(x1, r) ^ x0
    x0 = x0 + _KS2
    x1 = x1 + np.uint32(5)

    return x0 ^ x1


def _sample_kernel(logits_ref, out_ref, best_val, best_idx):
    j = pl.program_id(0)
    chunk_base = j * _CHUNK

    lane = jax.lax.broadcasted_iota(jnp.int32, (_SUB, _TILE), 1)
    row_iota = jax.lax.broadcasted_iota(jnp.int32, (_SUB, _TILE), 0) * _N

    def strip_body(s, _):
        row0 = s * _SUB
        rowoff = row_iota + (row0 * _N + 42)

        acc_val = jnp.full((_SUB, _TILE), _NEG_INF, jnp.float32)
        acc_idx = jnp.zeros((_SUB, _TILE), jnp.int32)

        for t in range(_NTILE):
            col = lane + (chunk_base + t * _TILE)
            x1 = (col + rowoff).astype(jnp.uint32)
            bits = _threefry_bits(x1)

            fbits = (bits >> np.uint32(9)) | np.uint32(0x3F800000)
            floats = (jax.lax.bitcast_convert_type(fbits, jnp.float32)
                      - np.float32(1.0))
            u = jnp.maximum(_TINY, floats + _TINY)
            neglog_u = -jnp.log(u)
            t4 = jnp.log(neglog_u)

            tile = logits_ref[pl.ds(row0, _SUB), pl.ds(t * _TILE, _TILE)]
            val = jnp.where(col < _N, tile - t4, _NEG_INF)

            upd = val > acc_val
            acc_val = jnp.maximum(acc_val, val)
            acc_idx = jnp.where(upd, col, acc_idx)

        m = jnp.max(acc_val, axis=1, keepdims=True)
        cand = jnp.where(acc_val == m, acc_idx, np.int32(2**31 - 1))
        idx = jnp.min(cand, axis=1, keepdims=True)

        rows = pl.ds(row0, _SUB)

        @pl.when(j == 0)
        def _init():
            best_val[rows, :] = m
            best_idx[rows, :] = idx

        @pl.when(j > 0)
        def _update():
            better = m > best_val[rows, :]
            best_val[rows, :] = jnp.where(better, m, best_val[rows, :])
            best_idx[rows, :] = jnp.where(better, idx, best_idx[rows, :])

        return 0

    jax.lax.fori_loop(0, _NSTRIP, strip_body, 0, unroll=False)

    @pl.when(j == _GRID - 1)
    def _finish():
        out_ref[...] = best_idx[...]


def kernel(logits):
    out = pl.pallas_call(
        _sample_kernel,
        grid=(_GRID,),
        in_specs=[pl.BlockSpec((_B, _CHUNK), lambda j: (0, j))],
        out_specs=pl.BlockSpec((_B, 1), lambda j: (0, 0)),
        out_shape=jax.ShapeDtypeStruct((_B, 1), jnp.int32),
        scratch_shapes=[
            pltpu.VMEM((_B, 1), jnp.float32),
            pltpu.VMEM((_B, 1), jnp.int32),
        ],
        compiler_params=pltpu.CompilerParams(
            dimension_semantics=("arbitrary",),
        ),
    )(logits)
    return out.reshape(_B)
